# baseline (speedup 1.0000x reference)
import jax
import jax.numpy as jnp
from jax.experimental import pallas as pl

_W = 50048


def _fuse_body(xyz_ref, scal_ref, rot_ref, op_ref, dc_ref, rest_ref, out_ref):
    out_ref[0:3, :] = xyz_ref[...]
    out_ref[3:6, :] = jnp.exp(scal_ref[...])
    q = rot_ref[...]
    norm = jnp.sqrt(jnp.sum(q * q, axis=0, keepdims=True))
    out_ref[6:10, :] = q / jnp.maximum(norm, 1e-12)
    out_ref[10:11, :] = 1.0 / (1.0 + jnp.exp(-op_ref[...]))
    out_ref[11:14, :] = dc_ref[:, 0, :]
    for i in range(15):
        out_ref[14 + 3 * i : 17 + 3 * i, :] = rest_ref[:, i, :]


def kernel(xyz, features_dc, features_rest, scaling, rotation, opacity):
    n = xyz.shape[0]
    xyz_t = xyz.T
    scal_t = scaling.T
    rot_t = rotation.T
    op_t = opacity.T
    dc_t = features_dc.transpose(2, 1, 0)
    rest_t = features_rest.transpose(2, 1, 0)

    grid = pl.cdiv(n, _W)

    def rows2(c):
        return pl.BlockSpec((c, _W), lambda i: (0, i))

    def rows3(c, m):
        return pl.BlockSpec((c, m, _W), lambda i: (0, 0, i))

    out = pl.pallas_call(
        _fuse_body,
        grid=(grid,),
        in_specs=[rows2(3), rows2(3), rows2(4), rows2(1), rows3(3, 1), rows3(3, 15)],
        out_specs=rows2(59),
        out_shape=jax.ShapeDtypeStruct((59, n), jnp.float32),
    )(xyz_t, scal_t, rot_t, op_t, dc_t, rest_t)
    return out.T


# speedup vs baseline: 1.0060x; 1.0060x over previous
import jax
import jax.numpy as jnp
from jax.experimental import pallas as pl

_W = 51200


def _fuse_body(xyz_ref, scal_ref, rot_ref, op_ref, dc_ref, rest_ref, out_ref):
    out_ref[0:3, :] = xyz_ref[...]
    out_ref[3:6, :] = jnp.exp(scal_ref[...])
    q = rot_ref[...]
    norm = jnp.sqrt(jnp.sum(q * q, axis=0, keepdims=True))
    out_ref[6:10, :] = q / jnp.maximum(norm, 1e-12)
    out_ref[10:11, :] = 1.0 / (1.0 + jnp.exp(-op_ref[...]))
    out_ref[11:14, :] = dc_ref[:, 0, :]
    for i in range(15):
        out_ref[14 + 3 * i : 17 + 3 * i, :] = rest_ref[:, i, :]


def kernel(xyz, features_dc, features_rest, scaling, rotation, opacity):
    n = xyz.shape[0]
    xyz_t = xyz.T
    scal_t = scaling.T
    rot_t = rotation.T
    op_t = opacity.T
    dc_t = features_dc.transpose(2, 1, 0)
    rest_t = features_rest.transpose(2, 1, 0)

    grid = pl.cdiv(n, _W)

    def rows2(c):
        return pl.BlockSpec((c, _W), lambda i: (0, i))

    def rows3(c, m):
        return pl.BlockSpec((c, m, _W), lambda i: (0, 0, i))

    out = pl.pallas_call(
        _fuse_body,
        grid=(grid,),
        in_specs=[rows2(3), rows2(3), rows2(4), rows2(1), rows3(3, 1), rows3(3, 15)],
        out_specs=rows2(59),
        out_shape=jax.ShapeDtypeStruct((59, n), jnp.float32),
    )(xyz_t, scal_t, rot_t, op_t, dc_t, rest_t)
    return out.T
